# baseline (device time: 35962 ns/iter reference)
import jax
import jax.numpy as jnp
from jax import lax
from jax.experimental import pallas as pl
from jax.experimental.pallas import tpu as pltpu

N_DEV = 4
EPS = 1e-5
GLOBAL_HW = 512 * 128


def kernel(x, Wp):
    b, h_per, w, c = x.shape
    c_out = Wp.shape[1]

    def body(x_ref, wp_ref, out_ref, own_ref, stats_ref, send_sems, recv_sems):
        my = lax.axis_index("i")

        xs = x_ref[...]
        s1 = jnp.sum(xs, axis=(1, 2))
        s2 = jnp.sum(xs * xs, axis=(1, 2))
        own_ref[0, :, :] = s1
        own_ref[1, :, :] = s2

        barrier_sem = pltpu.get_barrier_semaphore()
        for d in range(1, N_DEV):
            pl.semaphore_signal(
                barrier_sem, inc=1,
                device_id=((my + d) % N_DEV,),
                device_id_type=pl.DeviceIdType.MESH,
            )
        pl.semaphore_wait(barrier_sem, N_DEV - 1)

        for d in range(1, N_DEV):
            rdma = pltpu.make_async_remote_copy(
                src_ref=own_ref,
                dst_ref=stats_ref.at[d - 1],
                send_sem=send_sems.at[d - 1],
                recv_sem=recv_sems.at[d - 1],
                device_id=((my + d) % N_DEV,),
                device_id_type=pl.DeviceIdType.MESH,
            )
            rdma.start()

        for k in range(N_DEV - 1):
            recv = pltpu.make_async_remote_copy(
                src_ref=own_ref,
                dst_ref=stats_ref.at[k],
                send_sem=send_sems.at[k],
                recv_sem=recv_sems.at[k],
                device_id=(my,),
                device_id_type=pl.DeviceIdType.MESH,
            )
            recv.wait_recv()

        tot1 = s1 + stats_ref[0, 0] + stats_ref[1, 0] + stats_ref[2, 0]
        tot2 = s2 + stats_ref[0, 1] + stats_ref[1, 1] + stats_ref[2, 1]
        mean = tot1 / GLOBAL_HW
        var = tot2 / GLOBAL_HW - mean * mean
        inv = lax.rsqrt(var + EPS)

        hn = (xs - mean[:, None, None, :]) * inv[:, None, None, :]
        a = hn * jax.nn.sigmoid(hn)
        a2 = a.reshape(b * h_per * w, c).astype(jnp.bfloat16)
        wb = wp_ref[...].astype(jnp.bfloat16)
        y = jnp.dot(a2, wb, preferred_element_type=jnp.float32)
        out_ref[...] = y.reshape(b, h_per, w, c_out).astype(jnp.bfloat16)

        for k in range(N_DEV - 1):
            snd = pltpu.make_async_remote_copy(
                src_ref=own_ref,
                dst_ref=stats_ref.at[k],
                send_sem=send_sems.at[k],
                recv_sem=recv_sems.at[k],
                device_id=(my,),
                device_id_type=pl.DeviceIdType.MESH,
            )
            snd.wait_send()

    return pl.pallas_call(
        body,
        out_shape=jax.ShapeDtypeStruct((b, h_per, w, c_out), jnp.bfloat16),
        in_specs=[
            pl.BlockSpec(memory_space=pltpu.VMEM),
            pl.BlockSpec(memory_space=pltpu.VMEM),
        ],
        out_specs=pl.BlockSpec(memory_space=pltpu.VMEM),
        scratch_shapes=[
            pltpu.VMEM((2, b, c), jnp.float32),
            pltpu.VMEM((N_DEV - 1, 2, b, c), jnp.float32),
            pltpu.SemaphoreType.DMA((N_DEV - 1,)),
            pltpu.SemaphoreType.DMA((N_DEV - 1,)),
        ],
        compiler_params=pltpu.CompilerParams(collective_id=0),
    )(x, Wp)


# device time: 16678 ns/iter; 2.1563x vs baseline; 2.1563x over previous
import jax
import jax.numpy as jnp
from jax import lax
from jax.experimental import pallas as pl
from jax.experimental.pallas import tpu as pltpu

N_DEV = 4
EPS = 1e-5
GLOBAL_HW = 512 * 128


def kernel(x, Wp):
    xt = x.transpose(0, 1, 3, 2)
    b, h_per, c, w = xt.shape
    c_out = Wp.shape[1]

    def body(xt_ref, wp_ref, out_ref, own_ref, stats_ref, send_sems, recv_sems):
        my = lax.axis_index("i")

        xs = xt_ref[...]
        s1 = jnp.sum(xs, axis=(1, 3))
        s2 = jnp.sum(xs * xs, axis=(1, 3))
        own_ref[0, :, :] = s1
        own_ref[1, :, :] = s2

        barrier_sem = pltpu.get_barrier_semaphore()
        for d in range(1, N_DEV):
            pl.semaphore_signal(
                barrier_sem, inc=1,
                device_id=((my + d) % N_DEV,),
                device_id_type=pl.DeviceIdType.MESH,
            )
        pl.semaphore_wait(barrier_sem, N_DEV - 1)

        for d in range(1, N_DEV):
            rdma = pltpu.make_async_remote_copy(
                src_ref=own_ref,
                dst_ref=stats_ref.at[d - 1],
                send_sem=send_sems.at[d - 1],
                recv_sem=recv_sems.at[d - 1],
                device_id=((my + d) % N_DEV,),
                device_id_type=pl.DeviceIdType.MESH,
            )
            rdma.start()

        xb = xs.astype(jnp.bfloat16)

        for k in range(N_DEV - 1):
            recv = pltpu.make_async_remote_copy(
                src_ref=own_ref,
                dst_ref=stats_ref.at[k],
                send_sem=send_sems.at[k],
                recv_sem=recv_sems.at[k],
                device_id=(my,),
                device_id_type=pl.DeviceIdType.MESH,
            )
            recv.wait_recv()

        tot1 = s1 + stats_ref[0, 0] + stats_ref[1, 0] + stats_ref[2, 0]
        tot2 = s2 + stats_ref[0, 1] + stats_ref[1, 1] + stats_ref[2, 1]
        mean = tot1 / GLOBAL_HW
        var = tot2 / GLOBAL_HW - mean * mean
        inv = lax.rsqrt(var + EPS)
        mean_b = mean.astype(jnp.bfloat16)
        inv_b = inv.astype(jnp.bfloat16)

        hn = (xb - mean_b[:, None, :, None]) * inv_b[:, None, :, None]
        a = hn * jax.nn.sigmoid(hn)

        at = jnp.swapaxes(a.reshape(b * h_per, c, w), 1, 2)
        a2 = at.reshape(b * h_per * w, c)
        wb = wp_ref[...].astype(jnp.bfloat16)
        y = jnp.dot(a2, wb, preferred_element_type=jnp.float32)
        out_ref[...] = y.reshape(b, h_per, w, c_out).astype(jnp.bfloat16)

        for k in range(N_DEV - 1):
            snd = pltpu.make_async_remote_copy(
                src_ref=own_ref,
                dst_ref=stats_ref.at[k],
                send_sem=send_sems.at[k],
                recv_sem=recv_sems.at[k],
                device_id=(my,),
                device_id_type=pl.DeviceIdType.MESH,
            )
            snd.wait_send()

    return pl.pallas_call(
        body,
        out_shape=jax.ShapeDtypeStruct((b, h_per, w, c_out), jnp.bfloat16),
        in_specs=[
            pl.BlockSpec(memory_space=pltpu.VMEM),
            pl.BlockSpec(memory_space=pltpu.VMEM),
        ],
        out_specs=pl.BlockSpec(memory_space=pltpu.VMEM),
        scratch_shapes=[
            pltpu.VMEM((2, b, c), jnp.float32),
            pltpu.VMEM((N_DEV - 1, 2, b, c), jnp.float32),
            pltpu.SemaphoreType.DMA((N_DEV - 1,)),
            pltpu.SemaphoreType.DMA((N_DEV - 1,)),
        ],
        compiler_params=pltpu.CompilerParams(collective_id=0),
    )(xt, Wp)
